# baseline (speedup 1.0000x reference)
import functools

import jax
import jax.numpy as jnp
from jax import lax
from jax.experimental import pallas as pl
from jax.experimental.pallas import tpu as pltpu
from jax.experimental.pallas import tpu_sc as plsc

B = 16384
D = 64
NUM_NEG = 4
NSEG = 2 + NUM_NEG
V = 1000000
NC = 2
NS = 16
NW = NC * NS
L = 16

TILES = V // 128
TPW = 248
STRIDE = 244
WCOLS = TPW * 128
CW = 512
NCH = WCOLS // CW
TAIL_BASE = V - 128
TAIL_LO = TILES * 128

IDCH = 4096
MU = 1024
MI = 3584
EC = 512
OSLOTS = 8

_mesh = plsc.VectorSubcoreMesh(core_axis_name="c", subcore_axis_name="s")

_scratch = [
    pltpu.VMEM((D, CW), jnp.float32),
    pltpu.VMEM((D, CW), jnp.float32),
    pltpu.VMEM((IDCH,), jnp.int32),
    pltpu.VMEM((IDCH,), jnp.int32),
    pltpu.VMEM((MU,), jnp.int32),
    pltpu.VMEM((MU,), jnp.int32),
    pltpu.VMEM((MI,), jnp.int32),
    pltpu.VMEM((MI,), jnp.int32),
    pltpu.VMEM((EC,), jnp.int32),
    pltpu.VMEM((EC,), jnp.int32),
    pltpu.VMEM((OSLOTS, L, 2 * D), jnp.float32),
    pltpu.VMEM((D, 128), jnp.float32),
    pltpu.VMEM((D, 128), jnp.float32),
    pltpu.SemaphoreType.DMA,
    pltpu.SemaphoreType.DMA,
    pltpu.SemaphoreType.DMA,
    pltpu.SemaphoreType.DMA,
    pltpu.SemaphoreType.DMA((OSLOTS,)),
]


@functools.partial(
    pl.kernel,
    mesh=_mesh,
    out_type=jax.ShapeDtypeStruct((NSEG * B, 2 * D), jnp.float32),
    scratch_types=_scratch,
    compiler_params=pltpu.CompilerParams(
        use_tc_tiling_on_sc=True, needs_layout_passes=False),
)
def _stream_kernel(uw_t, iw_t, ut_t, it_t, u_ids, i_ids, ne_ids, out,
                   sbuf0, sbuf1, idb0, idb1, um, up, im, ip, ej, ep,
                   obuf, tailu, taili, ssem0, ssem1, isem, tsem, osem):
    sbuf = (sbuf0, sbuf1)
    ssem = (ssem0, ssem1)
    idb = (idb0, idb1)
    wid = lax.axis_index("s") * NC + lax.axis_index("c")
    iota = lax.iota(jnp.int32, L)

    start_tile = wid * STRIDE
    col_lo = pl.multiple_of(start_tile * 128, 128)
    col_hi = col_lo + WCOLS
    hi_eff = jnp.where(wid == NW - 1, jnp.int32(1 << 30), col_hi)

    tail_cp = [pltpu.async_copy(ut_t, tailu, tsem),
               pltpu.async_copy(it_t, taili, tsem)]

    def chunk_col(k):
        return pl.multiple_of(col_lo + k * CW, 128)

    stream_started = {}

    def start_stream(tab, k, par):
        stream_started[(id(tab), k)] = pltpu.async_copy(
            tab.at[:, pl.ds(chunk_col(k), CW)], sbuf[par], ssem[par])

    start_stream(uw_t, 0, 0)
    start_stream(uw_t, 1, 1)

    rounds = []
    for r in range(B // IDCH):
        rounds.append((u_ids, r * IDCH, r * IDCH, True))
    for r in range(B // IDCH):
        rounds.append((i_ids, r * IDCH, B + r * IDCH, False))
    for r in range(NUM_NEG * B // IDCH):
        rounds.append((ne_ids, r * IDCH, 2 * B + r * IDCH, False))

    id_cps = [None, None]
    ref0, off0, _, _ = rounds[0]
    id_cps[0] = pltpu.async_copy(ref0.at[pl.ds(off0, IDCH)], idb[0], isem)

    ucnt = jnp.int32(0)
    icnt = jnp.int32(0)
    for r, (ref, off, posbase, is_user) in enumerate(rounds):
        if r + 1 < len(rounds):
            nref, noff, _, _ = rounds[r + 1]
            id_cps[(r + 1) % 2] = pltpu.async_copy(
                nref.at[pl.ds(noff, IDCH)], idb[(r + 1) % 2], isem)
        id_cps[r % 2].wait()
        buf = idb[r % 2]
        mlist, plist = (um, up) if is_user else (im, ip)
        cap = (MU if is_user else MI) - L

        def sbody(g, cnt, buf=buf, posbase=posbase, mlist=mlist,
                  plist=plist, cap=cap):
            v = buf[pl.ds(g * L, L)]
            pos = posbase + g * L + iota
            m = (v >= col_lo) & (v < hi_eff)
            c = jnp.minimum(cnt, cap)
            plsc.store_compressed(mlist.at[pl.ds(c, L)], v, mask=m)
            plsc.store_compressed(plist.at[pl.ds(c, L)], pos, mask=m)
            return cnt + jnp.sum(m.astype(jnp.int32))

        if is_user:
            ucnt = lax.fori_loop(0, IDCH // L, sbody, ucnt)
        else:
            icnt = lax.fori_loop(0, IDCH // L, sbody, icnt)

    gtot = jnp.int32(0)

    def process_window(buf, wlo, wwidth, jbase, mlist, plist, mcnt, gtot):
        def rbody(j, ecnt):
            v = mlist[pl.ds(j * L, L)]
            p = plist[pl.ds(j * L, L)]
            valid = (j * L + iota) < mcnt
            m = valid & (v >= wlo) & (v < wlo + wwidth)
            e = jnp.minimum(ecnt, EC - L)
            plsc.store_compressed(ej.at[pl.ds(e, L)], v - jbase, mask=m)
            plsc.store_compressed(ep.at[pl.ds(e, L)], p, mask=m)
            return ecnt + jnp.sum(m.astype(jnp.int32))

        ecnt = lax.fori_loop(0, (mcnt + L - 1) >> 4, rbody, jnp.int32(0))

        def ebody(g, gtot):
            jv = ej[pl.ds(g * L, L)]
            pv = ep[pl.ds(g * L, L)]
            valid = (g * L + iota) < ecnt
            big = jnp.int32(1 << 30)
            j0 = jnp.min(jnp.where(iota == 0, jv, big))
            p0 = jnp.min(jnp.where(iota == 0, pv, big))
            jv = jnp.where(valid, jv, j0)
            pv = jnp.where(valid, pv, p0)
            slot = gtot & (OSLOTS - 1)
            dst = obuf.at[slot]

            @pl.when(gtot >= OSLOTS)
            def _():
                pltpu.make_async_copy(dst, out.at[pv], osem.at[slot]).wait()

            for c in range(D):
                cvec = jnp.full((L,), c, jnp.int32)
                val = plsc.load_gather(buf, [cvec, jv])
                plsc.store_scatter(dst, [iota, cvec], val)
            pltpu.async_copy(dst, out.at[pv], osem.at[slot])
            return gtot + 1

        return lax.fori_loop(0, (ecnt + L - 1) >> 4, ebody, gtot)

    for tab, mlist, plist, mcnt in ((uw_t, um, up, ucnt),
                                    (iw_t, im, ip, icnt)):
        if tab is iw_t:
            start_stream(iw_t, 0, 0)
            start_stream(iw_t, 1, 1)

        def cbody(i, gtot, tab=tab, mlist=mlist, plist=plist, mcnt=mcnt):
            for par in (0, 1):
                k = 2 * i + par
                pltpu.make_async_copy(
                    tab.at[:, pl.ds(chunk_col(k), CW)], sbuf[par],
                    ssem[par]).wait()
                gtot = process_window(
                    sbuf[par], chunk_col(k), CW, chunk_col(k), mlist, plist,
                    mcnt, gtot)
                kk = jnp.minimum(k + 2, NCH - 1)

                @pl.when(k + 2 < NCH)
                def _():
                    pltpu.async_copy(
                        tab.at[:, pl.ds(chunk_col(kk), CW)], sbuf[par],
                        ssem[par])
            return gtot

        gtot = lax.fori_loop(0, NCH // 2, cbody, gtot)
        if tail_cp:
            tail_cp.pop(0).wait()
            tail_cp.pop(0).wait()
        tbuf = tailu if tab is uw_t else taili
        gtot = process_window(tbuf, TAIL_LO, 128, TAIL_BASE, mlist, plist,
                              mcnt, gtot)

    def drain(g, _):
        slot = g & (OSLOTS - 1)
        pltpu.make_async_copy(
            obuf.at[slot], out.at[ep[pl.ds(0, L)]], osem.at[slot]).wait()
        return 0

    lax.fori_loop(jnp.maximum(gtot - OSLOTS, 0), gtot, drain, 0)


def kernel(user_ids, item_ids, ne_item_ids, user_weight, item_weight):
    uw_t = user_weight.T
    iw_t = item_weight.T
    ut_t = user_weight[TAIL_BASE:].T
    it_t = item_weight[TAIL_BASE:].T
    ne_flat = ne_item_ids.reshape(-1)
    wide = _stream_kernel(uw_t, iw_t, ut_t, it_t, user_ids, item_ids, ne_flat)
    return wide[:, :D]


# speedup vs baseline: 1.1057x; 1.1057x over previous
import functools

import jax
import jax.numpy as jnp
from jax import lax
from jax.experimental import pallas as pl
from jax.experimental.pallas import tpu as pltpu
from jax.experimental.pallas import tpu_sc as plsc

B = 16384
D = 64
NUM_NEG = 4
NSEG = 2 + NUM_NEG
V = 1000000
NC = 2
NS = 16
NW = NC * NS
L = 16

TILES = V // 128
TPW = 248
STRIDE = 244
WCOLS = TPW * 128
CW = 512
NCH = WCOLS // CW
TAIL_BASE = V - 128
TAIL_LO = TILES * 128

IDCH = 4096
MU = 1024
MI = 2048
EC = 512
OSLOTS = 8

_mesh = plsc.VectorSubcoreMesh(core_axis_name="c", subcore_axis_name="s")

_scratch = [
    pltpu.VMEM((D, CW), jnp.float32),
    pltpu.VMEM((D, CW), jnp.float32),
    pltpu.VMEM((IDCH,), jnp.int32),
    pltpu.VMEM((IDCH,), jnp.int32),
    pltpu.VMEM((MU,), jnp.int32),
    pltpu.VMEM((MU,), jnp.int32),
    pltpu.VMEM((MI,), jnp.int32),
    pltpu.VMEM((MI,), jnp.int32),
    pltpu.VMEM((MI,), jnp.int32),
    pltpu.VMEM((MI,), jnp.int32),
    pltpu.VMEM((EC,), jnp.int32),
    pltpu.VMEM((EC,), jnp.int32),
    pltpu.VMEM((OSLOTS, L, 2 * D), jnp.float32),
    pltpu.VMEM((D, 128), jnp.float32),
    pltpu.VMEM((D, 128), jnp.float32),
    pltpu.SemaphoreType.DMA,
    pltpu.SemaphoreType.DMA,
    pltpu.SemaphoreType.DMA,
    pltpu.SemaphoreType.DMA,
    pltpu.SemaphoreType.DMA((OSLOTS,)),
]


@functools.partial(
    pl.kernel,
    mesh=_mesh,
    out_type=jax.ShapeDtypeStruct((NSEG * B, 2 * D), jnp.float32),
    scratch_types=_scratch,
    compiler_params=pltpu.CompilerParams(
        use_tc_tiling_on_sc=True, needs_layout_passes=False),
)
def _stream_kernel(uw_t, iw_t, ut_t, it_t, u_ids, i_ids, ne_ids, out,
                   sbuf0, sbuf1, idb0, idb1, um, up, im0, ip0, im1, ip1, ej, ep,
                   obuf, tailu, taili, ssem0, ssem1, isem, tsem, osem):
    sbuf = (sbuf0, sbuf1)
    ssem = (ssem0, ssem1)
    idb = (idb0, idb1)
    wid = lax.axis_index("s") * NC + lax.axis_index("c")
    iota = lax.iota(jnp.int32, L)

    start_tile = wid * STRIDE
    col_lo = pl.multiple_of(start_tile * 128, 128)
    col_hi = col_lo + WCOLS
    mid = col_lo + 32 * CW
    hi_eff = jnp.where(wid == NW - 1, jnp.int32(1 << 30), col_hi)

    tail_cp = [pltpu.async_copy(ut_t, tailu, tsem),
               pltpu.async_copy(it_t, taili, tsem)]

    def chunk_col(k):
        return pl.multiple_of(col_lo + k * CW, 128)

    stream_started = {}

    def start_stream(tab, k, par):
        stream_started[(id(tab), k)] = pltpu.async_copy(
            tab.at[:, pl.ds(chunk_col(k), CW)], sbuf[par], ssem[par])

    start_stream(uw_t, 0, 0)
    start_stream(uw_t, 1, 1)

    rounds = []
    for r in range(B // IDCH):
        rounds.append((u_ids, r * IDCH, r * IDCH, True))
    for r in range(B // IDCH):
        rounds.append((i_ids, r * IDCH, B + r * IDCH, False))
    for r in range(NUM_NEG * B // IDCH):
        rounds.append((ne_ids, r * IDCH, 2 * B + r * IDCH, False))

    id_cps = [None, None]
    ref0, off0, _, _ = rounds[0]
    id_cps[0] = pltpu.async_copy(ref0.at[pl.ds(off0, IDCH)], idb[0], isem)

    ucnt = jnp.int32(0)
    icnt = (jnp.int32(0), jnp.int32(0))
    for r, (ref, off, posbase, is_user) in enumerate(rounds):
        if r + 1 < len(rounds):
            nref, noff, _, _ = rounds[r + 1]
            id_cps[(r + 1) % 2] = pltpu.async_copy(
                nref.at[pl.ds(noff, IDCH)], idb[(r + 1) % 2], isem)
        id_cps[r % 2].wait()
        buf = idb[r % 2]

        if is_user:
            def sbody(g, cnt, buf=buf, posbase=posbase):
                v = buf[pl.ds(g * L, L)]
                pos = posbase + g * L + iota
                m = (v >= col_lo) & (v < hi_eff)
                c = jnp.minimum(cnt, MU - L)
                plsc.store_compressed(um.at[pl.ds(c, L)], v, mask=m)
                plsc.store_compressed(up.at[pl.ds(c, L)], pos, mask=m)
                return cnt + jnp.sum(m.astype(jnp.int32))

            ucnt = lax.fori_loop(0, IDCH // L, sbody, ucnt)
        else:
            def sbody(g, carry, buf=buf, posbase=posbase):
                c0, c1 = carry
                v = buf[pl.ds(g * L, L)]
                pos = posbase + g * L + iota
                m0 = (v >= col_lo) & (v < mid)
                m1 = (v >= mid) & (v < hi_eff)
                e0 = jnp.minimum(c0, MI - L)
                e1 = jnp.minimum(c1, MI - L)
                plsc.store_compressed(im0.at[pl.ds(e0, L)], v, mask=m0)
                plsc.store_compressed(ip0.at[pl.ds(e0, L)], pos, mask=m0)
                plsc.store_compressed(im1.at[pl.ds(e1, L)], v, mask=m1)
                plsc.store_compressed(ip1.at[pl.ds(e1, L)], pos, mask=m1)
                return (c0 + jnp.sum(m0.astype(jnp.int32)),
                        c1 + jnp.sum(m1.astype(jnp.int32)))

            icnt = lax.fori_loop(0, IDCH // L, sbody, icnt)

    gtot = jnp.int32(0)

    def process_window(buf, wlo, wwidth, jbase, mlist, plist, mcnt, gtot):
        def rbody(j, ecnt):
            v = mlist[pl.ds(j * L, L)]
            p = plist[pl.ds(j * L, L)]
            valid = (j * L + iota) < mcnt
            m = valid & (v >= wlo) & (v < wlo + wwidth)
            e = jnp.minimum(ecnt, EC - L)
            plsc.store_compressed(ej.at[pl.ds(e, L)], v - jbase, mask=m)
            plsc.store_compressed(ep.at[pl.ds(e, L)], p, mask=m)
            return ecnt + jnp.sum(m.astype(jnp.int32))

        ecnt = lax.fori_loop(0, (mcnt + L - 1) >> 4, rbody, jnp.int32(0))

        def ebody(g, gtot):
            jv = ej[pl.ds(g * L, L)]
            pv = ep[pl.ds(g * L, L)]
            valid = (g * L + iota) < ecnt
            big = jnp.int32(1 << 30)
            j0 = jnp.min(jnp.where(iota == 0, jv, big))
            p0 = jnp.min(jnp.where(iota == 0, pv, big))
            jv = jnp.where(valid, jv, j0)
            pv = jnp.where(valid, pv, p0)
            slot = gtot & (OSLOTS - 1)
            dst = obuf.at[slot]

            @pl.when(gtot >= OSLOTS)
            def _():
                pltpu.make_async_copy(dst, out.at[pv], osem.at[slot]).wait()

            for c in range(D):
                cvec = jnp.full((L,), c, jnp.int32)
                val = plsc.load_gather(buf, [cvec, jv])
                plsc.store_scatter(dst, [iota, cvec], val)
            pltpu.async_copy(dst, out.at[pv], osem.at[slot])
            return gtot + 1

        return lax.fori_loop(0, (ecnt + L - 1) >> 4, ebody, gtot)

    segments = (
        (uw_t, 0, NCH // 2, um, up, ucnt, True),
        (iw_t, 0, 16, im0, ip0, icnt[0], True),
        (iw_t, 32, 15, im1, ip1, icnt[1], False),
    )
    for tab, k0, iters, mlist, plist, mcnt, prime in segments:
        if prime and tab is iw_t:
            start_stream(iw_t, 0, 0)
            start_stream(iw_t, 1, 1)

        def cbody(i, gtot, tab=tab, k0=k0, mlist=mlist, plist=plist,
                  mcnt=mcnt):
            for par in (0, 1):
                k = k0 + 2 * i + par
                pltpu.make_async_copy(
                    tab.at[:, pl.ds(chunk_col(k), CW)], sbuf[par],
                    ssem[par]).wait()
                gtot = process_window(
                    sbuf[par], chunk_col(k), CW, chunk_col(k), mlist, plist,
                    mcnt, gtot)
                kk = jnp.minimum(k + 2, NCH - 1)

                @pl.when(k + 2 < NCH)
                def _():
                    pltpu.async_copy(
                        tab.at[:, pl.ds(chunk_col(kk), CW)], sbuf[par],
                        ssem[par])
            return gtot

        gtot = lax.fori_loop(0, iters, cbody, gtot)
        if tab is uw_t:
            if tail_cp:
                tail_cp.pop(0).wait()
                tail_cp.pop(0).wait()
            gtot = process_window(tailu, TAIL_LO, 128, TAIL_BASE, um, up,
                                  ucnt, gtot)
    gtot = process_window(taili, TAIL_LO, 128, TAIL_BASE, im1, ip1,
                          icnt[1], gtot)

    def drain(g, _):
        slot = g & (OSLOTS - 1)
        pltpu.make_async_copy(
            obuf.at[slot], out.at[ep[pl.ds(0, L)]], osem.at[slot]).wait()
        return 0

    lax.fori_loop(jnp.maximum(gtot - OSLOTS, 0), gtot, drain, 0)


def kernel(user_ids, item_ids, ne_item_ids, user_weight, item_weight):
    uw_t = user_weight.T
    iw_t = item_weight.T
    ut_t = user_weight[TAIL_BASE:].T
    it_t = item_weight[TAIL_BASE:].T
    ne_flat = ne_item_ids.reshape(-1)
    wide = _stream_kernel(uw_t, iw_t, ut_t, it_t, user_ids, item_ids, ne_flat)
    return wide[:, :D]
